# Initial kernel scaffold; baseline (speedup 1.0000x reference)
import jax
import jax.numpy as jnp
from jax.experimental import pallas as pl


def kernel(ids_tensor, fixed_table, learned_table):
    raise NotImplementedError("write your pallas kernel here")


# speedup vs baseline: 2.4594x; 2.4594x over previous
import functools

import jax
import jax.numpy as jnp
from jax import lax
from jax.experimental import pallas as pl
from jax.experimental.pallas import tpu as pltpu
from jax.experimental.pallas import tpu_sc as plsc

DIM = 64
NUM_SPECIAL = 4
NC, NS = 2, 16
NW = NC * NS
CHUNK = 128


@functools.lru_cache(maxsize=None)
def _build(B):
    Bw = B // NW
    nchunk = Bw // CHUNK
    mesh = plsc.VectorSubcoreMesh(
        core_axis_name="c", subcore_axis_name="s", num_cores=NC, num_subcores=NS
    )

    @functools.partial(
        pl.kernel,
        out_type=jax.ShapeDtypeStruct((B, DIM), jnp.float32),
        mesh=mesh,
        compiler_params=pltpu.CompilerParams(
            use_tc_tiling_on_sc=False, needs_layout_passes=False
        ),
        scratch_types=[
            pltpu.VMEM((Bw,), jnp.int32),
            pltpu.VMEM((CHUNK,), jnp.int32),
            pltpu.VMEM((CHUNK, DIM), jnp.float32),
            pltpu.VMEM((NUM_SPECIAL, DIM), jnp.float32),
            pltpu.VMEM((NUM_SPECIAL, DIM), jnp.float32),
            pltpu.SemaphoreType.DMA,
        ],
    )
    def k(ids_hbm, fixed_hbm, learned_hbm, out_hbm,
          ids_v, fidx_v, rows_v, learned_v, ldiff_v, gsem):
        wid = lax.axis_index("s") * NC + lax.axis_index("c")
        base = pl.multiple_of(wid * Bw, 8)
        pltpu.sync_copy(ids_hbm.at[pl.ds(base, Bw)], ids_v)
        pltpu.sync_copy(learned_hbm, learned_v)
        for c in range(DIM // 16):
            l0c = learned_v[0, pl.ds(16 * c, 16)]
            for r in range(NUM_SPECIAL):
                ldiff_v[r, pl.ds(16 * c, 16)] = learned_v[r, pl.ds(16 * c, 16)] - l0c
        lb = [learned_v[0, pl.ds(16 * c, 16)] for c in range(DIM // 16)]
        iota16 = lax.iota(jnp.int32, 16)

        def chunk_body(g, _):
            off = g * CHUNK

            def fcomp(j, specv):
                v = ids_v[pl.ds(off + j * 16, 16)]
                isw = v >= NUM_SPECIAL
                fidx_v[pl.ds(j * 16, 16)] = jnp.where(isw, v - NUM_SPECIAL, 0)
                return specv | (~isw)

            specv = lax.fori_loop(
                0, CHUNK // 16, fcomp, jnp.zeros((16,), jnp.bool_)
            )
            nspec = plsc.all_reduce_population_count(specv)[0]
            pltpu.async_copy(fixed_hbm.at[fidx_v], rows_v, gsem).wait()

            def addrow(i4, _):
                for r in range(4):
                    i = i4 * 4 + r
                    for c in range(DIM // 16):
                        plsc.addupdate(rows_v.at[i, pl.ds(16 * c, 16)], lb[c])
                return 0

            lax.fori_loop(0, CHUNK // 4, addrow, 0)

            @pl.when(nspec > 0)
            def _fix():
                def fixg(j, _):
                    v = ids_v[pl.ds(off + j * 16, 16)]
                    m = v < NUM_SPECIAL

                    @pl.when(plsc.all_reduce_population_count(m)[0] > 0)
                    def _fix_group():
                        lid = jnp.where(m, v, 0)
                        ridx = j * 16 + iota16

                        def fixcol(col, _):
                            cv = jnp.full((16,), col, jnp.int32)
                            gv = plsc.load_gather(ldiff_v, [lid, cv])
                            plsc.addupdate_scatter(rows_v, [ridx, cv], gv, mask=m)
                            return 0

                        lax.fori_loop(0, DIM, fixcol, 0)

                    return 0

                lax.fori_loop(0, CHUNK // 16, fixg, 0)

            pltpu.sync_copy(rows_v, out_hbm.at[pl.ds(base + off, CHUNK)])
            return 0

        lax.fori_loop(0, nchunk, chunk_body, 0)

    return k


def kernel(ids_tensor, fixed_table, learned_table):
    Bt, H = ids_tensor.shape
    B = Bt * H
    ids_flat = ids_tensor.reshape(B).astype(jnp.int32)
    out = _build(B)(ids_flat, fixed_table, learned_table)
    return out.reshape(Bt, H, DIM)


# speedup vs baseline: 2.7597x; 1.1221x over previous
import functools

import jax
import jax.numpy as jnp
from jax import lax
from jax.experimental import pallas as pl
from jax.experimental.pallas import tpu as pltpu
from jax.experimental.pallas import tpu_sc as plsc

DIM = 64
NUM_SPECIAL = 4
NC, NS = 2, 16
NW = NC * NS
CHUNK = 128
NBUF = 4
GRP = CHUNK // 16


@functools.lru_cache(maxsize=None)
def _build(B):
    Bw = B // NW
    nchunk = Bw // CHUNK
    assert nchunk % NBUF == 0
    mesh = plsc.VectorSubcoreMesh(
        core_axis_name="c", subcore_axis_name="s", num_cores=NC, num_subcores=NS
    )

    @functools.partial(
        pl.kernel,
        out_type=jax.ShapeDtypeStruct((B, DIM), jnp.float32),
        mesh=mesh,
        compiler_params=pltpu.CompilerParams(
            use_tc_tiling_on_sc=False, needs_layout_passes=False
        ),
        scratch_types=[
            pltpu.VMEM((Bw,), jnp.int32),
            [pltpu.VMEM((CHUNK,), jnp.int32) for _ in range(NBUF)],
            [pltpu.VMEM((CHUNK, DIM), jnp.float32) for _ in range(NBUF)],
            pltpu.VMEM((NUM_SPECIAL, DIM), jnp.float32),
            pltpu.VMEM((NUM_SPECIAL, DIM), jnp.float32),
            [pltpu.SemaphoreType.DMA for _ in range(NBUF)],
            [pltpu.SemaphoreType.DMA for _ in range(NBUF)],
        ],
    )
    def k(ids_hbm, fixed_hbm, learned_hbm, out_hbm,
          ids_v, fidx, rows, learned_v, ldiff_v, gsem, wsem):
        wid = lax.axis_index("s") * NC + lax.axis_index("c")
        base = pl.multiple_of(wid * Bw, 8)
        pltpu.sync_copy(ids_hbm.at[pl.ds(base, Bw)], ids_v)
        pltpu.sync_copy(learned_hbm, learned_v)
        for c in range(DIM // 16):
            l0c = learned_v[0, pl.ds(16 * c, 16)]
            for r in range(NUM_SPECIAL):
                ldiff_v[r, pl.ds(16 * c, 16)] = learned_v[r, pl.ds(16 * c, 16)] - l0c
        lb = [learned_v[0, pl.ds(16 * c, 16)] for c in range(DIM // 16)]
        iota16 = lax.iota(jnp.int32, 16)

        def wb_descr(g, b):
            return pltpu.make_async_copy(
                rows[b], out_hbm.at[pl.ds(base + g * CHUNK, CHUNK)], wsem[b])

        def g_descr(b):
            return pltpu.make_async_copy(fixed_hbm.at[fidx[b]], rows[b], gsem[b])

        def launch(g, b):
            off = g * CHUNK
            for j in range(GRP):
                v = ids_v[pl.ds(off + j * 16, 16)]
                fidx[b][pl.ds(j * 16, 16)] = jnp.where(v >= NUM_SPECIAL,
                                                       v - NUM_SPECIAL, 0)
            pltpu.async_copy(fixed_hbm.at[fidx[b]], rows[b], gsem[b])

        def process(g, b):
            g_descr(b).wait()
            off = g * CHUNK

            def addrow(i8, _):
                for r in range(8):
                    i = i8 * 8 + r
                    for c in range(DIM // 16):
                        plsc.addupdate(rows[b].at[i, pl.ds(16 * c, 16)], lb[c])
                return 0

            lax.fori_loop(0, CHUNK // 8, addrow, 0)

            for j in range(GRP):
                v = ids_v[pl.ds(off + j * 16, 16)]
                m = v < NUM_SPECIAL

                @pl.when(plsc.all_reduce_population_count(m)[0] > 0)
                def _fix_group(j=j, v=v, m=m):
                    lid = jnp.where(m, v, 0)
                    ridx = j * 16 + iota16

                    def fixcol(col, _):
                        cv = jnp.full((16,), col, jnp.int32)
                        gv = plsc.load_gather(ldiff_v, [lid, cv])
                        plsc.addupdate_scatter(rows[b], [ridx, cv], gv, mask=m)
                        return 0

                    lax.fori_loop(0, DIM, fixcol, 0)

            pltpu.async_copy(rows[b], out_hbm.at[pl.ds(base + off, CHUNK)], wsem[b])

        for b in range(NBUF - 1):
            launch(jnp.int32(b), b)

        def body(p, _):
            for b in range(NBUF):
                g = p * NBUF + b
                gn = g + NBUF - 1
                bn = (b + NBUF - 1) % NBUF

                @pl.when(gn < nchunk)
                def _launch_next(gn=gn, bn=bn):
                    @pl.when(gn >= NBUF)
                    def _drain(gn=gn, bn=bn):
                        wb_descr(gn - NBUF, bn).wait()

                    launch(gn, bn)

                process(g, b)
            return 0

        lax.fori_loop(0, nchunk // NBUF, body, 0)

        for b in range(NBUF):
            wb_descr(jnp.int32(nchunk - NBUF + b), b).wait()

    return k


def kernel(ids_tensor, fixed_table, learned_table):
    Bt, H = ids_tensor.shape
    B = Bt * H
    ids_flat = ids_tensor.reshape(B).astype(jnp.int32)
    out = _build(B)(ids_flat, fixed_table, learned_table)
    return out.reshape(Bt, H, DIM)
